# baseline (speedup 1.0000x reference)
import functools

import jax
import jax.numpy as jnp
from jax import lax
from jax.experimental import pallas as pl
from jax.experimental.pallas import tpu as pltpu
from jax.experimental.pallas import tpu_sc as plsc

V = 27
D = 27
L = 16
NC = 2
NS = 16
NW = NC * NS

NI = 16384
NJ = 50
JT = 7
IB_ALL = NI // 128
IB_PW = IB_ALL // NW
ROWS_PW = 128 * IB_PW * NJ
STG = D * 1024
KSTRIDE = JT * 131072
OUT_WORDS = D * KSTRIDE
N_PH = JT * IB_PW


def _sc_gather(x_flat, w_flat):
    mesh = plsc.VectorSubcoreMesh(core_axis_name="c", subcore_axis_name="s")

    @functools.partial(
        pl.kernel,
        mesh=mesh,
        compiler_params=pltpu.CompilerParams(needs_layout_passes=False),
        out_type=jax.ShapeDtypeStruct((D, JT, 131072), jnp.float32),
        scratch_types=[
            pltpu.VMEM((V * D,), jnp.float32),
            pltpu.VMEM((ROWS_PW,), jnp.int32),
            pltpu.VMEM((D, 1, 1024), jnp.float32),
            pltpu.VMEM((D, 1, 1024), jnp.float32),
            pltpu.SemaphoreType.DMA,
            pltpu.SemaphoreType.DMA,
        ],
    )
    def k(x_hbm, w_hbm, out_hbm, w_v, xb_v, stg_a, stg_b, sem_a, sem_b):
        wid = lax.axis_index("s") * NC + lax.axis_index("c")
        pltpu.sync_copy(w_hbm, w_v)
        pltpu.sync_copy(x_hbm.at[pl.ds(wid * ROWS_PW, ROWS_PW)], xb_v)

        stgs = (stg_a, stg_b)
        sems = (sem_a, sem_b)
        str50 = lax.broadcasted_iota(jnp.int32, (L,), 0) * NJ
        ib0 = wid * IB_PW

        def out_dma(ph, buf):
            jt = ph >> 2
            ib = ib0 + (ph & 3)
            return pltpu.make_async_copy(
                stgs[buf],
                out_hbm.at[:, pl.ds(jt, 1), pl.ds(ib * 1024, 1024)],
                sems[buf])

        def compute(ph, buf):
            jt = ph >> 2
            ib = ph & 3
            jrc = jnp.where(jt == JT - 1, NJ - 8 * (JT - 1), 8)
            stg = stgs[buf]

            @plsc.parallel_loop(0, 8)
            def s_loop(s):
                base_i = (ib * 128 + s * 16) * NJ + jt * 8

                @plsc.parallel_loop(0, jrc)
                def jr_loop(jr):
                    xg = plsc.load_gather(xb_v, [str50 + (base_i + jr)])
                    wa = xg * D
                    sj = jr * 128 + s * 16
                    for kk in range(D):
                        wv = plsc.load_gather(w_v, [wa + kk])
                        stg[kk, 0, pl.ds(sj, L)] = wv

        def pair_body(p2, carry):
            for buf in range(2):
                ph = p2 * 2 + buf

                @pl.when(ph >= 2)
                def _drain():
                    out_dma(ph - 2, buf).wait()

                compute(ph, buf)
                out_dma(ph, buf).start()
            return carry

        lax.fori_loop(0, N_PH // 2, pair_body, 0)
        out_dma(N_PH - 2, 0).wait()
        out_dma(N_PH - 1, 1).wait()

    return k(x_flat, w_flat)


def kernel(x, W):
    assert x.shape == (NI, NJ) and W.shape == (V, D)
    x_flat = x.reshape(NI * NJ).astype(jnp.int32)
    out1 = _sc_gather(x_flat, W.astype(jnp.float32).reshape(V * D))
    a = out1.reshape(D, JT, 128, 8, 128)
    b = jnp.transpose(a, (2, 4, 1, 3, 0))
    c = b.reshape(NI, 8 * JT, D)
    return c[:, :NJ, :]


# speedup vs baseline: 1.7603x; 1.7603x over previous
import functools

import jax
import jax.numpy as jnp
from jax import lax
from jax.experimental import pallas as pl
from jax.experimental.pallas import tpu as pltpu
from jax.experimental.pallas import tpu_sc as plsc

V = 27
D = 27
L = 16
NC = 2
NS = 16
NW = NC * NS

NI = 16384
NJ = 50
JT = 7
JR_TAIL = NJ - 8 * (JT - 1)
IB_PW = (NI // 128) // NW
ROWS_PW = 128 * IB_PW * NJ
STG = D * 1024
KSTRIDE = JT * 131072
OUT_WORDS = D * KSTRIDE
N_PH = JT * IB_PW


def _sc_gather(x_flat, w_flat):
    mesh = plsc.VectorSubcoreMesh(core_axis_name="c", subcore_axis_name="s")

    @functools.partial(
        pl.kernel,
        mesh=mesh,
        compiler_params=pltpu.CompilerParams(needs_layout_passes=False),
        out_type=jax.ShapeDtypeStruct((OUT_WORDS,), jnp.float32),
        scratch_types=[
            pltpu.VMEM((V * D,), jnp.float32),
            pltpu.VMEM((ROWS_PW,), jnp.int32),
            pltpu.VMEM((STG,), jnp.float32),
            pltpu.VMEM((STG,), jnp.float32),
            pltpu.SemaphoreType.DMA,
            pltpu.SemaphoreType.DMA,
        ],
    )
    def k(x_hbm, w_hbm, out_hbm, w_v, xb_v, stg_a, stg_b, sem_a, sem_b):
        wid = lax.axis_index("s") * NC + lax.axis_index("c")
        pltpu.sync_copy(w_hbm, w_v)
        pltpu.sync_copy(x_hbm.at[pl.ds(wid * ROWS_PW, ROWS_PW)], xb_v)

        stgs = (stg_a, stg_b)
        sems = (sem_a, sem_b)
        str50 = lax.broadcasted_iota(jnp.int32, (L,), 0) * NJ
        ib0 = wid * IB_PW

        def dma(ph, buf, kk, n):
            jt = ph >> 2
            ib = ib0 + (ph & 3)
            dst = kk * KSTRIDE + jt * 131072 + ib * 1024
            return pltpu.make_async_copy(
                stgs[buf].at[pl.ds(kk * 1024, n)],
                out_hbm.at[pl.ds(dst, n)],
                sems[buf])

        def start(ph, buf, n):
            for kk in range(D):
                dma(ph, buf, kk, n).start()

        def drain(ph, buf, n):
            for kk in range(D):
                dma(ph, buf, kk, n).wait()

        def compute(ph, buf, jrc):
            jt = ph >> 2
            ib = ph & 3
            stg = stgs[buf]

            @plsc.parallel_loop(0, 8)
            def s_loop(s):
                base_i = (ib * 128 + s * 16) * NJ + jt * 8

                @plsc.parallel_loop(0, jrc)
                def jr_loop(jr):
                    xg = plsc.load_gather(xb_v, [str50 + (base_i + jr)])
                    wa = xg * D
                    sj = jr * 128 + s * 16
                    for kk in range(D):
                        wv = plsc.load_gather(w_v, [wa + kk])
                        stg[pl.ds(kk * 1024 + sj, L)] = wv

        compute(0, 0, 8)
        start(0, 0, 1024)
        compute(1, 1, 8)
        start(1, 1, 1024)

        def pair_body(p2, carry):
            for buf in range(2):
                ph = p2 * 2 + buf
                drain(ph - 2, buf, 1024)
                compute(ph, buf, 8)
                start(ph, buf, 1024)
            return carry

        lax.fori_loop(1, (N_PH - IB_PW) // 2, pair_body, 0)

        drain(N_PH - IB_PW - 2, 0, 1024)
        compute(N_PH - IB_PW, 0, JR_TAIL)
        start(N_PH - IB_PW, 0, JR_TAIL * 128)
        drain(N_PH - IB_PW - 1, 1, 1024)
        compute(N_PH - IB_PW + 1, 1, JR_TAIL)
        start(N_PH - IB_PW + 1, 1, JR_TAIL * 128)
        drain(N_PH - IB_PW, 0, JR_TAIL * 128)
        compute(N_PH - 2, 0, JR_TAIL)
        start(N_PH - 2, 0, JR_TAIL * 128)
        drain(N_PH - IB_PW + 1, 1, JR_TAIL * 128)
        compute(N_PH - 1, 1, JR_TAIL)
        start(N_PH - 1, 1, JR_TAIL * 128)
        drain(N_PH - 2, 0, JR_TAIL * 128)
        drain(N_PH - 1, 1, JR_TAIL * 128)

    return k(x_flat, w_flat)


def kernel(x, W):
    assert x.shape == (NI, NJ) and W.shape == (V, D)
    x_flat = x.reshape(NI * NJ).astype(jnp.int32)
    out1 = _sc_gather(x_flat, W.astype(jnp.float32).reshape(V * D))
    a = out1.reshape(D, JT, 128, 8, 128)
    b = jnp.transpose(a, (2, 4, 1, 3, 0))
    c = b.reshape(NI, 8 * JT, D)
    return c[:, :NJ, :]


# speedup vs baseline: 1.7697x; 1.0054x over previous
import functools

import jax
import jax.numpy as jnp
from jax import lax
from jax.experimental import pallas as pl
from jax.experimental.pallas import tpu as pltpu
from jax.experimental.pallas import tpu_sc as plsc

V = 27
D = 27
L = 16
NC = 2
NS = 16
NW = NC * NS

NI = 16384
NJ = 50
JT = 7
JR_TAIL = NJ - 8 * (JT - 1)
IB_PW = (NI // 128) // NW
ROWS_PW = 128 * IB_PW * NJ
STG = D * 1024
KSTRIDE = JT * 131072
OUT_WORDS = D * KSTRIDE
N_PH = JT * IB_PW


def _sc_gather(x_flat, w_flat):
    mesh = plsc.VectorSubcoreMesh(core_axis_name="c", subcore_axis_name="s")

    @functools.partial(
        pl.kernel,
        mesh=mesh,
        compiler_params=pltpu.CompilerParams(needs_layout_passes=False),
        out_type=jax.ShapeDtypeStruct((OUT_WORDS,), jnp.float32),
        scratch_types=[
            pltpu.VMEM((V * D,), jnp.float32),
            pltpu.VMEM((ROWS_PW,), jnp.int32),
            pltpu.VMEM((STG,), jnp.float32),
            pltpu.VMEM((STG,), jnp.float32),
            pltpu.SemaphoreType.DMA,
            pltpu.SemaphoreType.DMA,
        ],
    )
    def k(x_hbm, w_hbm, out_hbm, w_v, xb_v, stg_a, stg_b, sem_a, sem_b):
        wid = lax.axis_index("s") * NC + lax.axis_index("c")
        wcp = pltpu.make_async_copy(w_hbm, w_v, sem_a)
        xcp = pltpu.make_async_copy(
            x_hbm.at[pl.ds(wid * ROWS_PW, ROWS_PW)], xb_v, sem_b)
        wcp.start()
        xcp.start()
        wcp.wait()
        xcp.wait()

        stgs = (stg_a, stg_b)
        sems = (sem_a, sem_b)
        str50 = lax.broadcasted_iota(jnp.int32, (L,), 0) * NJ
        ib0 = wid * IB_PW

        def dma(ph, buf, kk, n):
            jt = ph >> 2
            ib = ib0 + (ph & 3)
            dst = kk * KSTRIDE + jt * 131072 + ib * 1024
            return pltpu.make_async_copy(
                stgs[buf].at[pl.ds(kk * 1024, n)],
                out_hbm.at[pl.ds(dst, n)],
                sems[buf])

        def start(ph, buf):
            tail = (ph >> 2) == JT - 1

            def sbody(kk, c2):
                @pl.when(jnp.logical_not(tail))
                def _full():
                    dma(ph, buf, kk, 1024).start()

                @pl.when(tail)
                def _short():
                    dma(ph, buf, kk, JR_TAIL * 128).start()
                return c2

            lax.fori_loop(0, D, sbody, 0)

        def drain(ph, buf):
            tail = (ph >> 2) == JT - 1

            def wbody(kk, c2):
                @pl.when(jnp.logical_not(tail))
                def _full():
                    dma(ph, buf, kk, 1024).wait()

                @pl.when(tail)
                def _short():
                    dma(ph, buf, kk, JR_TAIL * 128).wait()
                return c2

            lax.fori_loop(0, D, wbody, 0)

        def compute(ph, buf):
            jt = ph >> 2
            ib = ph & 3
            jrc = jnp.where(jt == JT - 1, JR_TAIL, 8)
            stg = stgs[buf]

            @plsc.parallel_loop(0, 8)
            def s_loop(s):
                base_i = (ib * 128 + s * 16) * NJ + jt * 8

                @plsc.parallel_loop(0, jrc)
                def jr_loop(jr):
                    xg = plsc.load_gather(xb_v, [str50 + (base_i + jr)])
                    wa = xg * D
                    sj = jr * 128 + s * 16
                    for kk in range(D):
                        wv = plsc.load_gather(w_v, [wa + kk])
                        stg[pl.ds(kk * 1024 + sj, L)] = wv

        def pair_body(p2, carry):
            for buf in range(2):
                ph = p2 * 2 + buf

                @pl.when(ph >= 2)
                def _drain():
                    drain(ph - 2, buf)

                compute(ph, buf)
                start(ph, buf)
            return carry

        lax.fori_loop(0, N_PH // 2, pair_body, 0)
        drain(N_PH - 2, 0)
        drain(N_PH - 1, 1)

    return k(x_flat, w_flat)


def kernel(x, W):
    assert x.shape == (NI, NJ) and W.shape == (V, D)
    x_flat = x.reshape(NI * NJ).astype(jnp.int32)
    out1 = _sc_gather(x_flat, W.astype(jnp.float32).reshape(V * D))
    a = out1.reshape(D, JT, 128, 8, 128)
    b = jnp.transpose(a, (2, 4, 1, 3, 0))
    c = b.reshape(NI, 8 * JT, D)
    return c[:, :NJ, :]
